# baseline (speedup 1.0000x reference)
import functools

import jax
import jax.numpy as jnp
from jax import lax
from jax.experimental import pallas as pl
from jax.experimental.pallas import tpu as pltpu
from jax.experimental.pallas import tpu_sc as plsc

B = 16384
D = 64
NC = 2
NS = 16
L = 16
NW = NC * NS
BPW = B // NW
CHUNK = 128
NCH = BPW // CHUNK


def _sc_body(h_hbm, r_hbm, t_hbm, ere_hbm, eim_hbm, rre_hbm, rim_hbm,
             out_hbm, hidx, ridx, tidx, hre, him, tre, tim, rre, rim,
             scores, sem):
    wid = lax.axis_index("s") * NC + lax.axis_index("c")
    pltpu.sync_copy(h_hbm.at[wid], hidx)
    pltpu.sync_copy(r_hbm.at[wid], ridx)
    pltpu.sync_copy(t_hbm.at[wid], tidx)
    for c in range(NCH):
        cps = [
            pltpu.async_copy(ere_hbm.at[hidx.at[c]], hre, sem),
            pltpu.async_copy(eim_hbm.at[hidx.at[c]], him, sem),
            pltpu.async_copy(ere_hbm.at[tidx.at[c]], tre, sem),
            pltpu.async_copy(eim_hbm.at[tidx.at[c]], tim, sem),
            pltpu.async_copy(rre_hbm.at[ridx.at[c]], rre, sem),
            pltpu.async_copy(rim_hbm.at[ridx.at[c]], rim, sem),
        ]
        for cp in cps:
            cp.wait()
        for g in range(CHUNK // L):
            rows = lax.iota(jnp.int32, L) + (g * L)

            def dim_step(d, acc, rows=rows):
                cols = jnp.full((L,), d, jnp.int32)
                a = plsc.load_gather(hre, [rows, cols])
                bb = plsc.load_gather(him, [rows, cols])
                cr = plsc.load_gather(rre, [rows, cols])
                ci = plsc.load_gather(rim, [rows, cols])
                e = plsc.load_gather(tre, [rows, cols])
                f = plsc.load_gather(tim, [rows, cols])
                return acc + e * (a * cr - bb * ci) + f * (bb * cr + a * ci)

            acc = lax.fori_loop(0, D, dim_step, jnp.zeros((L,), jnp.float32))
            scores[pl.ds(c * CHUNK + g * L, L)] = acc
    pltpu.sync_copy(scores, out_hbm.at[pl.ds(wid * BPW, BPW)])


@functools.partial(jax.jit)
def kernel(h, r, t, entity_re, entity_im, relation_re, relation_im):
    h3 = h.astype(jnp.int32).reshape(NW, NCH, CHUNK)
    r3 = r.astype(jnp.int32).reshape(NW, NCH, CHUNK)
    t3 = t.astype(jnp.int32).reshape(NW, NCH, CHUNK)
    mesh = plsc.VectorSubcoreMesh(
        core_axis_name="c", subcore_axis_name="s", num_cores=NC,
        num_subcores=NS)
    run = pl.kernel(
        _sc_body,
        out_type=jax.ShapeDtypeStruct((B,), jnp.float32),
        mesh=mesh,
        scratch_types=[
            pltpu.VMEM((NCH, CHUNK), jnp.int32),
            pltpu.VMEM((NCH, CHUNK), jnp.int32),
            pltpu.VMEM((NCH, CHUNK), jnp.int32),
            pltpu.VMEM((CHUNK, D), jnp.float32),
            pltpu.VMEM((CHUNK, D), jnp.float32),
            pltpu.VMEM((CHUNK, D), jnp.float32),
            pltpu.VMEM((CHUNK, D), jnp.float32),
            pltpu.VMEM((CHUNK, D), jnp.float32),
            pltpu.VMEM((CHUNK, D), jnp.float32),
            pltpu.VMEM((BPW,), jnp.float32),
            pltpu.SemaphoreType.DMA,
        ],
        compiler_params=pltpu.CompilerParams(
            needs_layout_passes=False, use_tc_tiling_on_sc=False),
    )
    return run(h3, r3, t3, entity_re, entity_im, relation_re, relation_im)


# speedup vs baseline: 2.1342x; 2.1342x over previous
import functools

import jax
import jax.numpy as jnp
from jax import lax
from jax.experimental import pallas as pl
from jax.experimental.pallas import tpu as pltpu
from jax.experimental.pallas import tpu_sc as plsc

B = 16384
D = 64
NC = 2
NS = 16
L = 16
NW = NC * NS
BPW = B // NW
CH = 32
NCH = BPW // CH


def _sc_body(h_hbm, r_hbm, t_hbm, ere_hbm, eim_hbm, rre_hbm, rim_hbm,
             out_hbm, hidx, ridx, tidx, bufsA, bufsB, scores, semA, semB):
    wid = lax.axis_index("s") * NC + lax.axis_index("c")
    pltpu.sync_copy(h_hbm.at[wid], hidx)
    pltpu.sync_copy(r_hbm.at[wid], ridx)
    pltpu.sync_copy(t_hbm.at[wid], tidx)

    def issue(c, bufs, sem):
        hre, him, tre, tim, rre, rim = bufs
        for g in range(CH // L):
            base = c * CH + g * L
            hv = hidx[pl.ds(base, L)]
            tv = tidx[pl.ds(base, L)]
            rv = ridx[pl.ds(base, L)]
            hj, hs = hv >> 3, hv & 7
            tj, ts = tv >> 3, tv & 7
            rj, rs = rv >> 3, rv & 7
            for k in range(L):
                p = g * L + k
                pltpu.async_copy(ere_hbm.at[hj[k], hs[k]], hre.at[p], sem)
                pltpu.async_copy(eim_hbm.at[hj[k], hs[k]], him.at[p], sem)
                pltpu.async_copy(ere_hbm.at[tj[k], ts[k]], tre.at[p], sem)
                pltpu.async_copy(eim_hbm.at[tj[k], ts[k]], tim.at[p], sem)
                pltpu.async_copy(rre_hbm.at[rj[k], rs[k]], rre.at[p], sem)
                pltpu.async_copy(rim_hbm.at[rj[k], rs[k]], rim.at[p], sem)

    def drain(bufs, sem):
        for buf in bufs:
            pltpu.make_async_copy(
                ere_hbm.at[pl.ds(0, CH), 0], buf, sem).wait()

    def compute(c, bufs):
        hre, him, tre, tim, rre, rim = bufs
        for g in range(CH // L):
            rows = lax.iota(jnp.int32, L) + (g * L)

            def dim_step(d, acc, rows=rows):
                cols = jnp.full((L,), d, jnp.int32)
                a = plsc.load_gather(hre, [rows, cols])
                bb = plsc.load_gather(him, [rows, cols])
                cr = plsc.load_gather(rre, [rows, cols])
                ci = plsc.load_gather(rim, [rows, cols])
                e = plsc.load_gather(tre, [rows, cols])
                f = plsc.load_gather(tim, [rows, cols])
                return acc + e * (a * cr - bb * ci) + f * (bb * cr + a * ci)

            acc = lax.fori_loop(0, D, dim_step, jnp.zeros((L,), jnp.float32))
            scores[pl.ds(c * CH + g * L, L)] = acc

    issue(0, bufsA, semA)

    def pair(m, carry):
        c0 = m * 2
        issue(c0 + 1, bufsB, semB)
        drain(bufsA, semA)
        compute(c0, bufsA)

        @pl.when(m < NCH // 2 - 1)
        def _():
            issue(c0 + 2, bufsA, semA)

        drain(bufsB, semB)
        compute(c0 + 1, bufsB)
        return carry

    lax.fori_loop(0, NCH // 2, pair, 0)
    pltpu.sync_copy(scores, out_hbm.at[pl.ds(wid * BPW, BPW)])


@functools.partial(jax.jit)
def kernel(h, r, t, entity_re, entity_im, relation_re, relation_im):
    h2 = h.astype(jnp.int32).reshape(NW, BPW)
    r2 = r.astype(jnp.int32).reshape(NW, BPW)
    t2 = t.astype(jnp.int32).reshape(NW, BPW)
    ere = entity_re.reshape(-1, 8, D)
    eim = entity_im.reshape(-1, 8, D)
    rre = relation_re.reshape(-1, 8, D)
    rim = relation_im.reshape(-1, 8, D)
    mesh = plsc.VectorSubcoreMesh(
        core_axis_name="c", subcore_axis_name="s", num_cores=NC,
        num_subcores=NS)
    row_bufs = [pltpu.VMEM((CH, D), jnp.float32) for _ in range(6)]
    run = pl.kernel(
        _sc_body,
        out_type=jax.ShapeDtypeStruct((B,), jnp.float32),
        mesh=mesh,
        scratch_types=[
            pltpu.VMEM((BPW,), jnp.int32),
            pltpu.VMEM((BPW,), jnp.int32),
            pltpu.VMEM((BPW,), jnp.int32),
            row_bufs,
            [pltpu.VMEM((CH, D), jnp.float32) for _ in range(6)],
            pltpu.VMEM((BPW,), jnp.float32),
            pltpu.SemaphoreType.DMA,
            pltpu.SemaphoreType.DMA,
        ],
        compiler_params=pltpu.CompilerParams(needs_layout_passes=False),
    )
    return run(h2, r2, t2, ere, eim, rre, rim)
